# Initial kernel scaffold; baseline (speedup 1.0000x reference)
import jax
import jax.numpy as jnp
from jax.experimental import pallas as pl


def kernel(x, edge_index, batch, embed, W1l, W1r, b1, W2l, W2r, b2, Wlin, blin):
    raise NotImplementedError("write your pallas kernel here")


# speedup vs baseline: 5.8740x; 5.8740x over previous
import functools

import jax
import jax.numpy as jnp
from jax import lax
from jax.experimental import pallas as pl
from jax.experimental.pallas import tpu as pltpu
from jax.experimental.pallas import tpu_sc as plsc

N = 100000
E = 1600000
V = 64
D = 32
H = 64
C = 2
G = 512

NC = 2
NS = 16
FC = 16
NQ = H // FC

TCB = 512
NPAD = 100352
NBLK_TC = NPAD // TCB
ROWS_PER_TILE = NPAD // NS

EBLK = 128
NB = 8
EPAD_BLOCKS = 12544
EPAD = EPAD_BLOCKS * EBLK
BLK_PER_TILE = EPAD_BLOCKS // NS
HALF_BLOCKS = EPAD_BLOCKS // NC
BLK_PER_TILE_HALF = HALF_BLOCKS // NS
DUMP_ROWS = 784


def _sc_mesh():
    return plsc.VectorSubcoreMesh(
        core_axis_name="c", subcore_axis_name="s",
        num_cores=NC, num_subcores=NS)


def _fill_rows(ref, nrows, value):
    v = jnp.full((FC,), value, jnp.float32)

    @pl.loop(0, nrows)
    def _(i):
        ref[i] = v


def _zero_slab(slab, zbuf, sid):
    base = sid * ROWS_PER_TILE

    @pl.loop(0, ROWS_PER_TILE // EBLK)
    def _(i):
        pltpu.sync_copy(zbuf, slab.at[pl.ds(base + i * EBLK, EBLK)])


def _dump_slab(slab, out_hbm, rows_v, sid):
    base = sid * ROWS_PER_TILE

    @pl.loop(0, ROWS_PER_TILE // DUMP_ROWS)
    def _(i):
        r0 = base + i * DUMP_ROWS
        tmp = rows_v.at[pl.ds(0, DUMP_ROWS)]
        pltpu.sync_copy(slab.at[pl.ds(r0, DUMP_ROWS)], tmp)
        pltpu.sync_copy(tmp, out_hbm.at[pl.ds(r0, DUMP_ROWS)])


def _ones_pass(dst_hbm, slab, idxd_v, ones_v, blk_lo, nblk_tile, sid):
    base_blk = blk_lo + sid * nblk_tile

    @pl.loop(0, nblk_tile // NB)
    def _(it):
        b0 = base_blk + it * NB
        pltpu.sync_copy(dst_hbm.at[pl.ds(b0, NB)], idxd_v)
        for j in range(NB):
            pltpu.sync_copy(ones_v, slab.at[idxd_v.at[j]], add=True)


def _gather_pass(src_hbm, dst_hbm, table_hbm, slab,
                 idxs_v, idxd_v, rows_v, sem, sid):
    base_blk = sid * BLK_PER_TILE

    @pl.loop(0, BLK_PER_TILE // NB)
    def _(it):
        b0 = base_blk + it * NB
        pltpu.sync_copy(src_hbm.at[pl.ds(b0, NB)], idxs_v)
        pltpu.sync_copy(dst_hbm.at[pl.ds(b0, NB)], idxd_v)
        descs = []
        for j in range(NB):
            descs.append(
                pltpu.async_copy(table_hbm.at[idxs_v.at[j]],
                                 rows_v.at[pl.ds(j * EBLK, EBLK)], sem))
        for d in descs:
            d.wait()
        for j in range(NB):
            pltpu.sync_copy(rows_v.at[pl.ds(j * EBLK, EBLK)],
                            slab.at[idxd_v.at[j]], add=True)


def _sc1_body(src_hbm, dst_hbm, t0, t1, t2, t3,
              deg0, deg1, a0, a1, a2, a3,
              slab, idxs_v, idxd_v, rows_v, ones_v, zbuf_v, sem):
    cid = lax.axis_index("c")
    sid = lax.axis_index("s")
    _fill_rows(ones_v, EBLK, 1.0)
    _fill_rows(zbuf_v, EBLK, 0.0)
    tables = [t0, t1, t2, t3]
    aggs = [a0, a1, a2, a3]
    degs = [deg0, deg1]

    for c in range(NC):
        @pl.when(cid == c)
        def _(c=c):
            _zero_slab(slab, zbuf_v, sid)
            plsc.subcore_barrier()
            _ones_pass(dst_hbm, slab, idxd_v, ones_v,
                       c * HALF_BLOCKS, BLK_PER_TILE_HALF, sid)
            plsc.subcore_barrier()
            _dump_slab(slab, degs[c], rows_v, sid)
            for q in (2 * c, 2 * c + 1):
                _zero_slab(slab, zbuf_v, sid)
                plsc.subcore_barrier()
                _gather_pass(src_hbm, dst_hbm, tables[q], slab,
                             idxs_v, idxd_v, rows_v, sem, sid)
                plsc.subcore_barrier()
                _dump_slab(slab, aggs[q], rows_v, sid)


def _sc2_body(src_hbm, dst_hbm, t0, t1, t2, t3,
              a0, a1, a2, a3,
              slab, idxs_v, idxd_v, rows_v, zbuf_v, sem):
    cid = lax.axis_index("c")
    sid = lax.axis_index("s")
    _fill_rows(zbuf_v, EBLK, 0.0)
    tables = [t0, t1, t2, t3]
    aggs = [a0, a1, a2, a3]

    for c in range(NC):
        @pl.when(cid == c)
        def _(c=c):
            for q in (2 * c, 2 * c + 1):
                _zero_slab(slab, zbuf_v, sid)
                plsc.subcore_barrier()
                _gather_pass(src_hbm, dst_hbm, tables[q], slab,
                             idxs_v, idxd_v, rows_v, sem, sid)
                plsc.subcore_barrier()
                _dump_slab(slab, aggs[q], rows_v, sid)


def _sc_agg1(src2d, dst2d, hl0):
    out = [jax.ShapeDtypeStruct((NPAD, FC), jnp.float32)] * 6
    scratch = [
        pltpu.VMEM_SHARED((NPAD, FC), jnp.float32),
        pltpu.VMEM((NB, EBLK), jnp.int32),
        pltpu.VMEM((NB, EBLK), jnp.int32),
        pltpu.VMEM((NB * EBLK, FC), jnp.float32),
        pltpu.VMEM((EBLK, FC), jnp.float32),
        pltpu.VMEM((EBLK, FC), jnp.float32),
        pltpu.SemaphoreType.DMA,
    ]
    fn = pl.kernel(_sc1_body, out_type=out, mesh=_sc_mesh(),
                   scratch_types=scratch,
                   compiler_params=pltpu.CompilerParams(
                       use_tc_tiling_on_sc=False))
    return fn(src2d, dst2d, *hl0)


def _sc_agg2(src2d, dst2d, h1):
    out = [jax.ShapeDtypeStruct((NPAD, FC), jnp.float32)] * 4
    scratch = [
        pltpu.VMEM_SHARED((NPAD, FC), jnp.float32),
        pltpu.VMEM((NB, EBLK), jnp.int32),
        pltpu.VMEM((NB, EBLK), jnp.int32),
        pltpu.VMEM((NB * EBLK, FC), jnp.float32),
        pltpu.VMEM((EBLK, FC), jnp.float32),
        pltpu.SemaphoreType.DMA,
    ]
    fn = pl.kernel(_sc2_body, out_type=out, mesh=_sc_mesh(),
                   scratch_types=scratch,
                   compiler_params=pltpu.CompilerParams(
                       use_tc_tiling_on_sc=False))
    return fn(src2d, dst2d, *h1)


def _tc1_body(x_ref, emb_ref, wl_ref, wr_ref,
              o0, o1, o2, o3, hr_ref):
    xb = x_ref[0, 0, :]
    onehot = (xb[:, None] ==
              lax.broadcasted_iota(jnp.int32, (TCB, V), 1)
              ).astype(jnp.float32)
    tl = jnp.dot(emb_ref[...], wl_ref[...],
                 preferred_element_type=jnp.float32)
    tr = jnp.dot(emb_ref[...], wr_ref[...],
                 preferred_element_type=jnp.float32)
    hl = jnp.dot(onehot, tl, preferred_element_type=jnp.float32)
    hr_ref[...] = jnp.dot(onehot, tr, preferred_element_type=jnp.float32)
    for q, r in enumerate((o0, o1, o2, o3)):
        r[...] = hl[:, q * FC:(q + 1) * FC]


def _tc_prep(x3d, embed, W1l, W1r):
    chunk_out = jax.ShapeDtypeStruct((NPAD, FC), jnp.float32)
    return pl.pallas_call(
        _tc1_body,
        grid=(NBLK_TC,),
        in_specs=[
            pl.BlockSpec((1, 1, TCB), lambda i: (i, 0, 0)),
            pl.BlockSpec((V, D), lambda i: (0, 0)),
            pl.BlockSpec((D, H), lambda i: (0, 0)),
            pl.BlockSpec((D, H), lambda i: (0, 0)),
        ],
        out_specs=[pl.BlockSpec((TCB, FC), lambda i: (i, 0))] * 4
        + [pl.BlockSpec((TCB, H), lambda i: (i, 0))],
        out_shape=[chunk_out] * 4
        + [jax.ShapeDtypeStruct((NPAD, H), jnp.float32)],
    )(x3d, embed, W1l, W1r)


def _tc2_body(a0, a1, a2, a3, d0, d1, hr_ref, b1_ref,
              o0, o1, o2, o3):
    deg = jnp.maximum(d0[:, 0:1] + d1[:, 0:1], 1.0)
    deginv = 1.0 / deg
    for q, (ar, orf) in enumerate(zip((a0, a1, a2, a3), (o0, o1, o2, o3))):
        z = (ar[...] * deginv
             + hr_ref[:, q * FC:(q + 1) * FC]
             + b1_ref[0, q * FC:(q + 1) * FC][None, :])
        orf[...] = jnp.maximum(z, 0.0)


def _tc_relu1(a1q, deg0, deg1, hr0, b1_2d):
    chunk_spec = pl.BlockSpec((TCB, FC), lambda i: (i, 0))
    return pl.pallas_call(
        _tc2_body,
        grid=(NBLK_TC,),
        in_specs=[chunk_spec] * 6 + [
            pl.BlockSpec((TCB, H), lambda i: (i, 0)),
            pl.BlockSpec((1, H), lambda i: (0, 0)),
        ],
        out_specs=[chunk_spec] * 4,
        out_shape=[jax.ShapeDtypeStruct((NPAD, FC), jnp.float32)] * 4,
    )(*a1q, deg0, deg1, hr0, b1_2d)


def _tc3_body(a0, a1, a2, a3, g0, g1, g2, g3, d0, d1,
              w2l_ref, w2r_ref, b2_ref, batch_ref,
              pooled_ref, cnt_ref):
    i = pl.program_id(0)

    @pl.when(i == 0)
    def _():
        pooled_ref[...] = jnp.zeros_like(pooled_ref)
        cnt_ref[...] = jnp.zeros_like(cnt_ref)

    deginv = 1.0 / jnp.maximum(d0[:, 0:1] + d1[:, 0:1], 1.0)
    agg = jnp.concatenate(
        [a0[...], a1[...], a2[...], a3[...]], axis=1) * deginv
    h1 = jnp.concatenate([g0[...], g1[...], g2[...], g3[...]], axis=1)
    z = (jnp.dot(agg, w2l_ref[...], preferred_element_type=jnp.float32)
         + jnp.dot(h1, w2r_ref[...], preferred_element_type=jnp.float32)
         + b2_ref[...])
    h2 = jnp.maximum(z, 0.0)
    bb = batch_ref[0, 0, :]
    onehot_t = (lax.broadcasted_iota(jnp.int32, (G, TCB), 0) ==
                bb[None, :]).astype(jnp.float32)
    pooled_ref[...] += jnp.dot(onehot_t, h2,
                               preferred_element_type=jnp.float32)
    cnt_ref[...] += jnp.dot(onehot_t, jnp.ones((TCB, H), jnp.float32),
                            preferred_element_type=jnp.float32)


def _tc_layer2_pool(a2q, h1q, deg0, deg1, W2l, W2r, b2_2d, batch3d):
    chunk_spec = pl.BlockSpec((TCB, FC), lambda i: (i, 0))
    return pl.pallas_call(
        _tc3_body,
        grid=(NBLK_TC,),
        in_specs=[chunk_spec] * 10 + [
            pl.BlockSpec((H, H), lambda i: (0, 0)),
            pl.BlockSpec((H, H), lambda i: (0, 0)),
            pl.BlockSpec((1, H), lambda i: (0, 0)),
            pl.BlockSpec((1, 1, TCB), lambda i: (i, 0, 0)),
        ],
        out_specs=[
            pl.BlockSpec((G, H), lambda i: (0, 0)),
            pl.BlockSpec((G, H), lambda i: (0, 0)),
        ],
        out_shape=[
            jax.ShapeDtypeStruct((G, H), jnp.float32),
            jax.ShapeDtypeStruct((G, H), jnp.float32),
        ],
    )(*a2q, *h1q, deg0, deg1, W2l, W2r, b2_2d, batch3d)


def _tc4_body(pooled_ref, cnt_ref, wlin_ref, blin_ref, out_ref):
    pm = pooled_ref[...] / jnp.maximum(cnt_ref[...], 1.0)
    out_ref[...] = (jnp.dot(pm, wlin_ref[...],
                            preferred_element_type=jnp.float32)
                    + blin_ref[...])


def _tc_head(pooled, cnt, Wlin, blin_2d):
    return pl.pallas_call(
        _tc4_body,
        out_shape=jax.ShapeDtypeStruct((G, C), jnp.float32),
    )(pooled, cnt, Wlin, blin_2d)


@jax.jit
def _run(x, edge_index, batch, embed, W1l, W1r, b1, W2l, W2r, b2,
         Wlin, blin):
    x = x.astype(jnp.int32)
    batch = batch.astype(jnp.int32)
    src = edge_index[0].astype(jnp.int32)
    dst = edge_index[1].astype(jnp.int32)

    src2d = jnp.concatenate(
        [src, jnp.zeros((EPAD - E,), jnp.int32)]).reshape(EPAD_BLOCKS, EBLK)
    dst2d = jnp.concatenate(
        [dst, jnp.full((EPAD - E,), N, jnp.int32)]).reshape(EPAD_BLOCKS, EBLK)
    x3d = jnp.concatenate(
        [x, jnp.zeros((NPAD - N,), jnp.int32)]).reshape(NBLK_TC, 1, TCB)
    batch3d = jnp.concatenate(
        [batch, jnp.full((NPAD - N,), G, jnp.int32)]).reshape(NBLK_TC, 1, TCB)
    b1_2d = b1.reshape(1, H)
    b2_2d = b2.reshape(1, H)
    blin_2d = blin.reshape(1, C)

    *hl0q, hr0 = _tc_prep(x3d, embed, W1l, W1r)
    deg0, deg1, *a1q = _sc_agg1(src2d, dst2d, hl0q)
    h1q = _tc_relu1(a1q, deg0, deg1, hr0, b1_2d)
    a2q = _sc_agg2(src2d, dst2d, h1q)
    pooled, cnt = _tc_layer2_pool(a2q, h1q, deg0, deg1, W2l, W2r,
                                  b2_2d, batch3d)
    return _tc_head(pooled, cnt, Wlin, blin_2d)


def kernel(x, edge_index, batch, embed, W1l, W1r, b1, W2l, W2r, b2,
           Wlin, blin):
    return _run(x, edge_index, batch, embed, W1l, W1r, b1, W2l, W2r, b2,
                Wlin, blin)
